# baseline (speedup 1.0000x reference)
import jax
import jax.numpy as jnp
from jax.experimental import pallas as pl

_BM = 32


def _tc_body(s_ref, y_ref, wb_ref, ws_ref, c_ref, o_ref):
    y = y_ref[...]
    s = s_ref[...]
    wb = wb_ref[...]
    ws = ws_ref[...]
    c = c_ref[...]
    nn, f = y.shape[2], y.shape[3]
    fs = ws.shape[2]
    ysm = y[:, :, :, :fs]
    p0 = jnp.sum(y * wb[0:1].reshape(1, 1, nn, f), axis=(2, 3))
    p1 = jnp.sum(y * wb[1:2].reshape(1, 1, nn, f), axis=(2, 3))
    p2 = jnp.sum(ysm * ws[0:1].reshape(1, 1, nn, fs), axis=(2, 3))
    p3 = jnp.sum(ysm * ws[1:2].reshape(1, 1, nn, fs), axis=(2, 3))
    e01 = jnp.where(s == 1, p1, p0)
    e23 = jnp.where(s == 3, p3, p2)
    e = jnp.where(s < 2, e01, e23)
    c01 = jnp.where(s == 1, c[1:2, 0:1], c[0:1, 0:1])
    c23 = jnp.where(s == 3, c[3:4, 0:1], c[2:3, 0:1])
    ca = jnp.where(s < 2, c01, c23)
    o_ref[...] = jnp.sum(e + ca, axis=1, keepdims=True)


def kernel(species, y, W_big, b_big, W_small, b_small, self_energies):
    b, a, nn, f = y.shape
    fs = W_small.shape[-1]
    inv = 1.0 / nn
    wb = (jnp.transpose(W_big, (1, 0, 2)) * inv).astype(jnp.float32)
    ws = (jnp.transpose(W_small, (1, 0, 2)) * inv).astype(jnp.float32)
    c_tab = (jnp.concatenate([jnp.sum(b_big, 0), jnp.sum(b_small, 0)], 0) * inv
             + self_energies).reshape(4, 1).astype(jnp.float32)

    out = pl.pallas_call(
        _tc_body,
        grid=(b // _BM,),
        in_specs=[
            pl.BlockSpec((_BM, a), lambda i: (i, 0)),
            pl.BlockSpec((_BM, a, nn, f), lambda i: (i, 0, 0, 0)),
            pl.BlockSpec((2, nn, f), lambda i: (0, 0, 0)),
            pl.BlockSpec((2, nn, fs), lambda i: (0, 0, 0)),
            pl.BlockSpec((4, 1), lambda i: (0, 0)),
        ],
        out_specs=pl.BlockSpec((_BM, 1), lambda i: (i, 0)),
        out_shape=jax.ShapeDtypeStruct((b, 1), jnp.float32),
    )(species.astype(jnp.int32), y, wb, ws, c_tab)
    return out.reshape(b)


# speedup vs baseline: 1.8456x; 1.8456x over previous
import jax
import jax.numpy as jnp
from jax.experimental import pallas as pl
from jax.experimental.pallas import tpu as pltpu

_BM = 8


def _tc_body(s_smem, sv_ref, y_ref, w_ref, c_ref, o_ref):
    a = s_smem.shape[1]
    nn = w_ref.shape[1]
    sa = sv_ref[...]
    c = c_ref[...]
    c01 = jnp.where(sa == 1, c[1:2, 0:1], c[0:1, 0:1])
    c23 = jnp.where(sa == 3, c[3:4, 0:1], c[2:3, 0:1])
    ca = jnp.where(sa < 2, c01, c23)
    casum = jnp.sum(ca, axis=1, keepdims=True)
    for m in range(s_smem.shape[0]):
        acc = jnp.zeros((nn, w_ref.shape[2]), jnp.float32)
        for at in range(a):
            sv = s_smem[m, at]
            w = w_ref[pl.ds(sv, 1), :, :][0]
            acc = acc + y_ref[m, pl.ds(at * nn, nn), :] * w
        o_ref[m:m + 1, 0:1] = jnp.sum(acc).reshape(1, 1) + casum[m:m + 1, 0:1]


def kernel(species, y, W_big, b_big, W_small, b_small, self_energies):
    b, a, nn, f = y.shape
    fs = W_small.shape[-1]
    inv = 1.0 / nn
    wb = jnp.transpose(W_big, (1, 0, 2))
    ws = jnp.pad(jnp.transpose(W_small, (1, 0, 2)),
                 ((0, 0), (0, 0), (0, f - fs)))
    w_tab = (jnp.concatenate([wb, ws], axis=0) * inv).astype(jnp.float32)
    c_tab = (jnp.concatenate([jnp.sum(b_big, 0), jnp.sum(b_small, 0)], 0) * inv
             + self_energies).reshape(4, 1).astype(jnp.float32)
    y2 = y.reshape(b, a * nn, f)
    s32 = species.astype(jnp.int32)

    out = pl.pallas_call(
        _tc_body,
        grid=(b // _BM,),
        in_specs=[
            pl.BlockSpec((_BM, a), lambda i: (i, 0), memory_space=pltpu.SMEM),
            pl.BlockSpec((_BM, a), lambda i: (i, 0)),
            pl.BlockSpec((_BM, a * nn, f), lambda i: (i, 0, 0)),
            pl.BlockSpec((4, nn, f), lambda i: (0, 0, 0)),
            pl.BlockSpec((4, 1), lambda i: (0, 0)),
        ],
        out_specs=pl.BlockSpec((_BM, 1), lambda i: (i, 0)),
        out_shape=jax.ShapeDtypeStruct((b, 1), jnp.float32),
    )(s32, s32, y2, w_tab, c_tab)
    return out.reshape(b)


# speedup vs baseline: 6.6309x; 3.5929x over previous
import jax
import jax.numpy as jnp
from jax.experimental import pallas as pl

_BB = 128
_BA = 16


def _tc_body(s_ref, y_ref, w_ref, c_ref, o_ref):
    j = pl.program_id(1)
    nn, f = y_ref.shape[1], y_ref.shape[2]
    s = s_ref[...]
    acc = jnp.zeros((f, _BB), jnp.float32)
    for i in range(nn):
        w0 = w_ref[0, i]
        w1 = w_ref[1, i]
        w2 = w_ref[2, i]
        w3 = w_ref[3, i]
        for a in range(_BA):
            sa = s[a:a + 1, :]
            wlo = jnp.where(sa == 1, w1, w0)
            whi = jnp.where(sa == 3, w3, w2)
            wsel = jnp.where(sa >= 2, whi, wlo)
            acc = acc + y_ref[a, i] * wsel
    main = jnp.sum(acc, axis=0)
    c = c_ref[...]
    c01 = jnp.where(s == 1, c[1:2, 0:1], c[0:1, 0:1])
    c23 = jnp.where(s == 3, c[3:4, 0:1], c[2:3, 0:1])
    ca = jnp.where(s >= 2, c23, c01)
    val = (main + jnp.sum(ca, axis=0)).reshape(1, 1, _BB)

    @pl.when(j == 0)
    def _init():
        o_ref[...] = val

    @pl.when(j > 0)
    def _accum():
        o_ref[...] = o_ref[...] + val


def kernel(species, y, W_big, b_big, W_small, b_small, self_energies):
    b, a, nn, f = y.shape
    fs = W_small.shape[-1]
    inv = 1.0 / nn
    wb = jnp.transpose(W_big, (1, 0, 2))
    ws = jnp.pad(jnp.transpose(W_small, (1, 0, 2)),
                 ((0, 0), (0, 0), (0, f - fs)))
    w_tab = (jnp.concatenate([wb, ws], axis=0) * inv).astype(jnp.float32)
    w_bcast = jnp.broadcast_to(w_tab[:, :, :, None], (4, nn, f, _BB))
    c_tab = (jnp.concatenate([jnp.sum(b_big, 0), jnp.sum(b_small, 0)], 0) * inv
             + self_energies).reshape(4, 1).astype(jnp.float32)
    y_t = jnp.transpose(y, (1, 2, 3, 0))
    s_t = jnp.transpose(species.astype(jnp.int32), (1, 0))

    out = pl.pallas_call(
        _tc_body,
        grid=(b // _BB, a // _BA),
        in_specs=[
            pl.BlockSpec((_BA, _BB), lambda i, j: (j, i)),
            pl.BlockSpec((_BA, nn, f, _BB), lambda i, j: (j, 0, 0, i)),
            pl.BlockSpec((4, nn, f, _BB), lambda i, j: (0, 0, 0, 0)),
            pl.BlockSpec((4, 1), lambda i, j: (0, 0)),
        ],
        out_specs=pl.BlockSpec((1, 1, _BB), lambda i, j: (i, 0, 0)),
        out_shape=jax.ShapeDtypeStruct((b // _BB, 1, _BB), jnp.float32),
    )(s_t, y_t, w_bcast, c_tab)
    return out.reshape(b)


# speedup vs baseline: 7.0053x; 1.0565x over previous
import jax
import jax.numpy as jnp
from jax.experimental import pallas as pl

_BB = 128
_BA = 32


def _tc_body(s_ref, y_ref, w_ref, c_ref, o_ref):
    j = pl.program_id(1)
    nn, f = y_ref.shape[1], y_ref.shape[2]
    s = s_ref[...]
    acc = jnp.zeros((f, _BB), jnp.float32)
    for i in range(nn):
        w0 = w_ref[0, i]
        w1 = w_ref[1, i]
        w2 = w_ref[2, i]
        w3 = w_ref[3, i]
        for a in range(_BA):
            sa = s[a:a + 1, :]
            wlo = jnp.where(sa == 1, w1, w0)
            whi = jnp.where(sa == 3, w3, w2)
            wsel = jnp.where(sa >= 2, whi, wlo)
            acc = acc + y_ref[a, i] * wsel
    main = jnp.sum(acc, axis=0)
    c = c_ref[...]
    c01 = jnp.where(s == 1, c[1:2, 0:1], c[0:1, 0:1])
    c23 = jnp.where(s == 3, c[3:4, 0:1], c[2:3, 0:1])
    ca = jnp.where(s >= 2, c23, c01)
    val = (main + jnp.sum(ca, axis=0)).reshape(1, 1, _BB)

    @pl.when(j == 0)
    def _init():
        o_ref[...] = val

    @pl.when(j > 0)
    def _accum():
        o_ref[...] = o_ref[...] + val


def kernel(species, y, W_big, b_big, W_small, b_small, self_energies):
    b, a, nn, f = y.shape
    fs = W_small.shape[-1]
    inv = 1.0 / nn
    wb = jnp.transpose(W_big, (1, 0, 2))
    ws = jnp.pad(jnp.transpose(W_small, (1, 0, 2)),
                 ((0, 0), (0, 0), (0, f - fs)))
    w_tab = (jnp.concatenate([wb, ws], axis=0) * inv).astype(jnp.float32)
    w_bcast = jnp.broadcast_to(w_tab[:, :, :, None], (4, nn, f, _BB))
    c_tab = (jnp.concatenate([jnp.sum(b_big, 0), jnp.sum(b_small, 0)], 0) * inv
             + self_energies).reshape(4, 1).astype(jnp.float32)
    y_t = jnp.transpose(y, (1, 2, 3, 0))
    s_t = jnp.transpose(species.astype(jnp.int32), (1, 0))

    out = pl.pallas_call(
        _tc_body,
        grid=(b // _BB, a // _BA),
        in_specs=[
            pl.BlockSpec((_BA, _BB), lambda i, j: (j, i)),
            pl.BlockSpec((_BA, nn, f, _BB), lambda i, j: (j, 0, 0, i)),
            pl.BlockSpec((4, nn, f, _BB), lambda i, j: (0, 0, 0, 0)),
            pl.BlockSpec((4, 1), lambda i, j: (0, 0)),
        ],
        out_specs=pl.BlockSpec((1, 1, _BB), lambda i, j: (i, 0, 0)),
        out_shape=jax.ShapeDtypeStruct((b // _BB, 1, _BB), jnp.float32),
    )(s_t, y_t, w_bcast, c_tab)
    return out.reshape(b)
